# baseline (speedup 1.0000x reference)
import functools

import jax
import jax.numpy as jnp
from jax import lax
from jax.experimental import pallas as pl
from jax.experimental.pallas import tpu as pltpu
from jax.experimental.pallas import tpu_sc as plsc

NC = 2
NS = 16
L = 16
NW = NC * NS
NBUF = 4


def kernel(global_info, x, ptr):
    B, D = global_info.shape
    TOTAL = x.shape[0]
    rows_per_w = TOTAL // NW
    C = L
    nchunks = rows_per_w // C

    mesh = plsc.VectorSubcoreMesh(core_axis_name="c", subcore_axis_name="s")

    @functools.partial(
        pl.kernel,
        out_type=jax.ShapeDtypeStruct((TOTAL, 2 * D), jnp.float32),
        mesh=mesh,
        scratch_types=[
            pltpu.VMEM((L,), jnp.int32),
            [pltpu.VMEM((C, D), jnp.float32) for _ in range(NBUF)],
            [pltpu.SemaphoreType.DMA for _ in range(NBUF)],
            [pltpu.SemaphoreType.DMA for _ in range(NBUF)],
            pltpu.SemaphoreType.DMA,
        ],
    )
    def run(g_hbm, x_hbm, ptr_hbm, out_hbm, ptr_v, bg, si, so, sx):
        wid = lax.axis_index("s") * NC + lax.axis_index("c")
        base = wid * rows_per_w

        xcopy = pltpu.async_copy(
            x_hbm.at[pl.ds(base, rows_per_w)],
            out_hbm.at[pl.ds(base, rows_per_w), pl.ds(0, D)], sx)

        pltpu.sync_copy(ptr_hbm.at[pl.ds(0, L)], ptr_v)
        pv = ptr_v[...]
        ones = jnp.full((L,), 1, jnp.int32)
        zeros = jnp.zeros((L,), jnp.int32)
        pbs = [
            pv.at[jnp.full((L,), b, jnp.int32)].get(mode="promise_in_bounds")
            for b in range(1, B)
        ]

        def seg_of(row0):
            rows = row0 + lax.iota(jnp.int32, L)
            seg = zeros
            for pb in pbs:
                seg = seg + jnp.where(pb <= rows, ones, zeros)
            return seg

        def start_in(k, j):
            pltpu.async_copy(g_hbm.at[seg_of(base + k * C)], bg[j], si[j])

        def wait_in(j):
            pltpu.make_async_copy(x_hbm.at[pl.ds(0, C)], bg[j], si[j]).wait()

        def start_out(k, j):
            row0 = base + k * C
            pltpu.async_copy(bg[j], out_hbm.at[pl.ds(row0, C), pl.ds(D, D)],
                             so[j])

        def wait_out(j):
            pltpu.make_async_copy(bg[j], out_hbm.at[pl.ds(0, C), pl.ds(D, D)],
                                  so[j]).wait()

        start_in(0, 0)
        start_in(1, 1)

        def step(k4, carry):
            for j in range(NBUF):
                k = k4 * NBUF + j
                wait_in(j)
                start_out(k, j)

                @pl.when(k >= 2)
                def _():
                    wait_out((j + 2) % NBUF)

                @pl.when(k + 2 < nchunks)
                def _():
                    start_in(k + 2, (j + 2) % NBUF)
            return carry

        lax.fori_loop(0, nchunks // NBUF, step, 0)
        wait_out((nchunks - 2) % NBUF)
        wait_out((nchunks - 1) % NBUF)
        xcopy.wait()

    return run(global_info, x, ptr)


# speedup vs baseline: 8.2549x; 8.2549x over previous
import functools

import jax
import jax.numpy as jnp
from jax import lax
from jax.experimental import pallas as pl
from jax.experimental.pallas import tpu as pltpu
from jax.experimental.pallas import tpu_sc as plsc

NC = 2
NS = 16
L = 16
NW = NC * NS
NBUF = 2
C = 16


def kernel(global_info, x, ptr):
    B, D = global_info.shape
    TOTAL = x.shape[0]
    rows_per_w = TOTAL // NW
    nchunks = rows_per_w // C

    mesh = plsc.VectorSubcoreMesh(core_axis_name="c", subcore_axis_name="s")

    @functools.partial(
        pl.kernel,
        out_type=jax.ShapeDtypeStruct((TOTAL, 2 * D), jnp.float32),
        mesh=mesh,
        scratch_types=[
            pltpu.VMEM((L,), jnp.int32),
            [pltpu.VMEM((C, 2 * D), jnp.float32) for _ in range(NBUF)],
            [pltpu.SemaphoreType.DMA for _ in range(NBUF)],
            [pltpu.SemaphoreType.DMA for _ in range(NBUF)],
            [pltpu.SemaphoreType.DMA for _ in range(NBUF)],
        ],
    )
    def run(g_hbm, x_hbm, ptr_hbm, out_hbm, ptr_v, bufs, si, sg, so):
        wid = lax.axis_index("s") * NC + lax.axis_index("c")
        base = wid * rows_per_w

        pltpu.sync_copy(ptr_hbm.at[pl.ds(0, L)], ptr_v)
        pv = ptr_v[...]
        ones = jnp.full((L,), 1, jnp.int32)
        zeros = jnp.zeros((L,), jnp.int32)
        pbs = [
            pv.at[jnp.full((L,), b, jnp.int32)].get(mode="promise_in_bounds")
            for b in range(1, B)
        ]

        def seg_of(row0):
            rows = row0 + lax.iota(jnp.int32, L)
            seg = zeros
            for pb in pbs:
                seg = seg + jnp.where(pb <= rows, ones, zeros)
            return seg

        def start_in(k, j):
            row0 = base + k * C
            pltpu.async_copy(x_hbm.at[pl.ds(row0, C)],
                             bufs[j].at[:, pl.ds(0, D)], si[j])

        def wait_in(j):
            pltpu.make_async_copy(x_hbm.at[pl.ds(0, C)],
                                  bufs[j].at[:, pl.ds(0, D)], si[j]).wait()

        def start_out(k, j):
            row0 = base + k * C
            pltpu.async_copy(bufs[j], out_hbm.at[pl.ds(row0, C)], so[j])

        def wait_out(j):
            pltpu.make_async_copy(bufs[j], out_hbm.at[pl.ds(0, C)],
                                  so[j]).wait()

        start_in(0, 0)

        def step(kb, carry):
            for j in range(NBUF):
                k = kb * NBUF + j
                seg = seg_of(base + k * C)
                wait_in(j)
                pltpu.async_copy(g_hbm.at[seg],
                                 bufs[j].at[:, pl.ds(D, D)], sg[j])
                pltpu.make_async_copy(x_hbm.at[pl.ds(0, C)],
                                      bufs[j].at[:, pl.ds(D, D)],
                                      sg[j]).wait()
                start_out(k, j)

                @pl.when(k >= NBUF - 1)
                def _():
                    wait_out((j + 1) % NBUF)

                @pl.when(k + 1 < nchunks)
                def _():
                    start_in(k + 1, (j + 1) % NBUF)
            return carry

        lax.fori_loop(0, nchunks // NBUF, step, 0)
        for i in range(1, NBUF):
            wait_out((nchunks - i) % NBUF)

    return run(global_info, x, ptr)


# speedup vs baseline: 23.3068x; 2.8234x over previous
import functools

import jax
import jax.numpy as jnp
from jax import lax
from jax.experimental import pallas as pl
from jax.experimental.pallas import tpu as pltpu
from jax.experimental.pallas import tpu_sc as plsc

NC = 2
NS = 16
L = 16
NW = NC * NS
NBUF = 2
C = 16


def kernel(global_info, x, ptr):
    B, D = global_info.shape
    TOTAL = x.shape[0]
    rows_per_w = TOTAL // NW
    nchunks = rows_per_w // C

    mesh = plsc.VectorSubcoreMesh(core_axis_name="c", subcore_axis_name="s")

    @functools.partial(
        pl.kernel,
        out_type=jax.ShapeDtypeStruct((TOTAL, 2 * D), jnp.float32),
        mesh=mesh,
        scratch_types=[
            pltpu.VMEM((L,), jnp.int32),
            [pltpu.VMEM((C, 2 * D), jnp.float32) for _ in range(NBUF)],
            [pltpu.SemaphoreType.DMA for _ in range(NBUF)],
            [pltpu.SemaphoreType.DMA for _ in range(NBUF)],
            [pltpu.SemaphoreType.DMA for _ in range(NBUF)],
        ],
    )
    def run(g_hbm, x_hbm, ptr_hbm, out_hbm, ptr_v, bufs, si, sg, so):
        wid = lax.axis_index("s") * NC + lax.axis_index("c")
        base = wid * rows_per_w

        pltpu.sync_copy(ptr_hbm.at[pl.ds(0, L)], ptr_v)
        pv = ptr_v[...]
        ones = jnp.full((L,), 1, jnp.int32)
        zeros = jnp.zeros((L,), jnp.int32)
        pbs = [
            pv.at[jnp.full((L,), b, jnp.int32)].get(mode="promise_in_bounds")
            for b in range(1, B)
        ]

        def seg_of(row0):
            rows = row0 + lax.iota(jnp.int32, L)
            seg = zeros
            for pb in pbs:
                seg = seg + jnp.where(pb <= rows, ones, zeros)
            return seg

        def start_in(k, j):
            row0 = base + k * C
            pltpu.async_copy(x_hbm.at[pl.ds(row0, C)],
                             bufs[j].at[:, pl.ds(0, D)], si[j])

        def wait_in(j):
            pltpu.make_async_copy(x_hbm.at[pl.ds(0, C)],
                                  bufs[j].at[:, pl.ds(0, D)], si[j]).wait()

        def start_out(k, j):
            row0 = base + k * C
            pltpu.async_copy(bufs[j], out_hbm.at[pl.ds(row0, C)], so[j])

        def wait_out(j):
            pltpu.make_async_copy(bufs[j], out_hbm.at[pl.ds(0, C)],
                                  so[j]).wait()

        start_in(0, 0)

        def step(kb, carry):
            cur = list(carry)
            for j in range(NBUF):
                k = kb * NBUF + j
                seg = seg_of(base + k * C)
                s0 = seg[0]
                s1 = seg[L - 1]
                need = jnp.logical_or(s0 != s1, cur[j] != s0)
                wait_in(j)

                @pl.when(need)
                def _():
                    pltpu.async_copy(g_hbm.at[seg],
                                     bufs[j].at[:, pl.ds(D, D)], sg[j])
                    pltpu.make_async_copy(x_hbm.at[pl.ds(0, C)],
                                          bufs[j].at[:, pl.ds(D, D)],
                                          sg[j]).wait()

                cur[j] = jnp.where(s0 == s1, s0, jnp.int32(-1))
                start_out(k, j)

                @pl.when(k >= NBUF - 1)
                def _():
                    wait_out((j + 1) % NBUF)

                @pl.when(k + 1 < nchunks)
                def _():
                    start_in(k + 1, (j + 1) % NBUF)
            return tuple(cur)

        lax.fori_loop(0, nchunks // NBUF, step,
                      tuple(jnp.int32(-1) for _ in range(NBUF)))
        for i in range(1, NBUF):
            wait_out((nchunks - i) % NBUF)

    return run(global_info, x, ptr)
